# Initial kernel scaffold; baseline (speedup 1.0000x reference)
import jax
import jax.numpy as jnp
from jax.experimental import pallas as pl


def kernel(image, depth, T, R, K, Kinv):
    raise NotImplementedError("write your pallas kernel here")


# speedup vs baseline: 35.8080x; 35.8080x over previous
import functools

import jax
import jax.numpy as jnp
from jax import lax
from jax.experimental import pallas as pl
from jax.experimental.pallas import tpu as pltpu
from jax.experimental.pallas import tpu_sc as plsc

B, C, H, W = 8, 3, 512, 512
HW = H * W
RB = 16
NQ = 4
QCELLS = HW // NQ
QROWS = H // NQ
TCH = 8192
_HUGE = 0x7FFFFFFF


def _bfr(x):
    u = lax.bitcast_convert_type(x, jnp.uint32)
    r = (u + jnp.uint32(0x7FFF) + ((u >> 16) & jnp.uint32(1))) & jnp.uint32(0xFFFF0000)
    return lax.bitcast_convert_type(r, jnp.float32)


def _proj_tc(a_ref, k_ref, t3_ref, d_ref, o_ref):
    b = pl.program_id(0)
    i = pl.program_id(1)
    gx = _bfr((i * RB + lax.broadcasted_iota(jnp.int32, (1, RB, W), 1)).astype(jnp.float32))
    gy = _bfr(lax.broadcasted_iota(jnp.int32, (1, RB, W), 2).astype(jnp.float32))
    d = d_ref[...]
    td0 = _bfr(t3_ref[b, 0] / d)
    td1 = _bfr(t3_ref[b, 1] / d)
    td2 = _bfr(t3_ref[b, 2] / d)
    kt0 = (k_ref[0, 0] * td0 + k_ref[0, 1] * td1) + k_ref[0, 2] * td2
    kt1 = (k_ref[1, 0] * td0 + k_ref[1, 1] * td1) + k_ref[1, 2] * td2
    kt2 = (k_ref[2, 0] * td0 + k_ref[2, 1] * td1) + k_ref[2, 2] * td2
    n0 = ((a_ref[b, 0] * gx + a_ref[b, 1] * gy) + a_ref[b, 2]) + kt0
    n1 = ((a_ref[b, 3] * gx + a_ref[b, 4] * gy) + a_ref[b, 5]) + kt1
    dn = ((a_ref[b, 6] * gx + a_ref[b, 7] * gy) + a_ref[b, 8]) + kt2
    p0 = jnp.clip(n0 / dn, 0, H - 1).astype(jnp.int32)
    p1 = jnp.clip(n1 / dn, 0, H - 1).astype(jnp.int32)
    o_ref[...] = p0 * W + p1


def _project(A, K, T, depth):
    return pl.pallas_call(
        _proj_tc,
        grid=(B, H // RB),
        in_specs=[
            pl.BlockSpec(memory_space=pltpu.SMEM),
            pl.BlockSpec(memory_space=pltpu.SMEM),
            pl.BlockSpec(memory_space=pltpu.SMEM),
            pl.BlockSpec((1, RB, W), lambda b, i: (b, i, 0)),
        ],
        out_specs=pl.BlockSpec((1, RB, W), lambda b, i: (b, i, 0)),
        out_shape=jax.ShapeDtypeStruct((B, H, W), jnp.int32),
    )(A, K, T, depth)


def _sc_body(t_hbm, img_hbm, out_hbm, nmax_v, tbuf_v,
             idx0_v, idx1_v, idx2_v, val0_v, val1_v, val2_v,
             orow0_v, orow1_v, orow2_v, key_v, row_v, gsem):
    idx_v = (idx0_v, idx1_v, idx2_v)
    val_v = (val0_v, val1_v, val2_v)
    orow_v = (orow0_v, orow1_v, orow2_v)
    cid = lax.axis_index("c")
    sid = lax.axis_index("s")
    wid = sid * 2 + cid
    b = wid // NQ
    q = wid % NQ
    lo = q * QCELLS
    lanes = lax.iota(jnp.int32, 16)
    nxt_idx = jnp.minimum(lanes + 1, 15)
    pair_idx = lanes - (lanes & 1)

    def init_body(i, _):
        nmax_v[pl.ds(i * 16, 16)] = jnp.full((16,), -1, jnp.int32)
        return _

    lax.fori_loop(0, QCELLS // 16, init_body, None, unroll=4)

    def chunk_body(ch, _):
        pltpu.sync_copy(t_hbm.at[pl.ds(b * HW + ch * TCH, TCH)], tbuf_v)

        def vec_body(v, __):
            tv = tbuf_v[pl.ds(v * 16, 16)]
            nvec = ch * TCH + v * 16 + lanes
            m = (tv >= lo) & (tv < lo + QCELLS)
            ukey = jnp.where(m, (tv - lo) * 16 + lanes, _HUGE)
            skey, sval = plsc.sort_key_val(ukey, nvec)
            key_v[...] = skey
            nxt = plsc.load_gather(key_v, [nxt_idx])
            win = ((skey >> 4) != (nxt >> 4)) | (lanes == 15)
            mask = win & (skey != _HUGE)
            plsc.store_scatter(nmax_v, [skey >> 4], sval, mask=mask)
            return __

        lax.fori_loop(0, TCH // 16, vec_body, None)
        return _

    lax.fori_loop(0, HW // TCH, chunk_body, None)

    boff = b * (C * HW)
    row0 = q * QROWS

    def pair_body(p, _):
        r = row0 + 2 * p

        def bld(v, __):
            nm = nmax_v[pl.ds(p * 1024 + v * 16, 16)]
            gcell = lo + p * 1024 + v * 16 + lanes
            safe = jnp.where(nm >= 0, nm, gcell)
            for c in range(C):
                idx_v[c][pl.ds(v * 16, 16)] = safe + (boff + c * HW)
            return __

        lax.fori_loop(0, 64, bld, None)
        cps = [
            pltpu.async_copy(img_hbm.at[idx_v[c]], val_v[c], gsem)
            for c in range(C)
        ]
        for cp in cps:
            cp.wait()

        for c in range(C):
            def vmax_body(j, __, c=c):
                nm_t = nmax_v[pl.ds(p * 1024 + j * 16, 16)]
                nm_b = nmax_v[pl.ds(p * 1024 + 512 + j * 16, 16)]
                top = jnp.where(nm_t >= 0, val_v[c][pl.ds(j * 16, 16)], 0.0)
                bot = jnp.where(nm_b >= 0, val_v[c][pl.ds(512 + j * 16, 16)], 0.0)
                row_v[pl.ds(j * 16, 16)] = jnp.maximum(top, bot)
                return __

            lax.fori_loop(0, W // 16, vmax_body, None)

            def hmax_body(j, __, c=c):
                base = j * 16
                a = plsc.load_gather(row_v, [base + pair_idx])
                bb = plsc.load_gather(row_v, [base + pair_idx + 1])
                orow_v[c][pl.ds(base, 16)] = jnp.maximum(a, bb)
                return __

            lax.fori_loop(0, W // 16, hmax_body, None)

        for c in range(C):
            o = boff + c * HW + r * W
            pltpu.sync_copy(orow_v[c], out_hbm.at[pl.ds(o, W)])
            pltpu.sync_copy(orow_v[c], out_hbm.at[pl.ds(o + W, W)])
        return _

    lax.fori_loop(0, QROWS // 2, pair_body, None)


@functools.cache
def _make_sc_call():
    return pl.kernel(
        _sc_body,
        out_type=jax.ShapeDtypeStruct((B * C * HW,), jnp.float32),
        mesh=plsc.VectorSubcoreMesh(core_axis_name="c", subcore_axis_name="s"),
        compiler_params=pltpu.CompilerParams(needs_layout_passes=False),
        scratch_types=[
        pltpu.VMEM((QCELLS,), jnp.int32),
        pltpu.VMEM((TCH,), jnp.int32),
        pltpu.VMEM((1024,), jnp.int32),
        pltpu.VMEM((1024,), jnp.int32),
        pltpu.VMEM((1024,), jnp.int32),
        pltpu.VMEM((1024,), jnp.float32),
        pltpu.VMEM((1024,), jnp.float32),
        pltpu.VMEM((1024,), jnp.float32),
        pltpu.VMEM((W,), jnp.float32),
        pltpu.VMEM((W,), jnp.float32),
        pltpu.VMEM((W,), jnp.float32),
        pltpu.VMEM((16,), jnp.int32),
        pltpu.VMEM((W,), jnp.float32),
        pltpu.SemaphoreType.DMA,
        ],
    )


def kernel(image, depth, T, R, K, Kinv):
    A = jnp.einsum('ij,bjk,kl->bil', K, R, Kinv)
    t = _project(_bfr(A).reshape(B, 9), _bfr(K), T.reshape(B, 3), depth)
    out = _make_sc_call()(t.reshape(B * HW), image.reshape(B * C * HW))
    return out.reshape(B, C, H, W)


# speedup vs baseline: 46.8441x; 1.3082x over previous
import functools

import jax
import jax.numpy as jnp
from jax import lax
from jax.experimental import pallas as pl
from jax.experimental.pallas import tpu as pltpu
from jax.experimental.pallas import tpu_sc as plsc

B, C, H, W = 8, 3, 512, 512
HW = H * W
RB = 16
NQ = 4
QCELLS = HW // NQ
QROWS = H // NQ
TCH = 8192
_HUGE = 0x7FFFFFFF


def _bfr(x):
    u = lax.bitcast_convert_type(x, jnp.uint32)
    r = (u + jnp.uint32(0x7FFF) + ((u >> 16) & jnp.uint32(1))) & jnp.uint32(0xFFFF0000)
    return lax.bitcast_convert_type(r, jnp.float32)


def _proj_tc(a_ref, k_ref, t3_ref, d_ref, o_ref):
    b = pl.program_id(0)
    i = pl.program_id(1)
    gx = _bfr((i * RB + lax.broadcasted_iota(jnp.int32, (1, RB, W), 1)).astype(jnp.float32))
    gy = _bfr(lax.broadcasted_iota(jnp.int32, (1, RB, W), 2).astype(jnp.float32))
    d = d_ref[...]
    td0 = _bfr(t3_ref[b, 0] / d)
    td1 = _bfr(t3_ref[b, 1] / d)
    td2 = _bfr(t3_ref[b, 2] / d)
    kt0 = (k_ref[0, 0] * td0 + k_ref[0, 1] * td1) + k_ref[0, 2] * td2
    kt1 = (k_ref[1, 0] * td0 + k_ref[1, 1] * td1) + k_ref[1, 2] * td2
    kt2 = (k_ref[2, 0] * td0 + k_ref[2, 1] * td1) + k_ref[2, 2] * td2
    n0 = ((a_ref[b, 0] * gx + a_ref[b, 1] * gy) + a_ref[b, 2]) + kt0
    n1 = ((a_ref[b, 3] * gx + a_ref[b, 4] * gy) + a_ref[b, 5]) + kt1
    dn = ((a_ref[b, 6] * gx + a_ref[b, 7] * gy) + a_ref[b, 8]) + kt2
    p0 = jnp.clip(n0 / dn, 0, H - 1).astype(jnp.int32)
    p1 = jnp.clip(n1 / dn, 0, H - 1).astype(jnp.int32)
    o_ref[...] = p0 * W + p1


def _project(A, K, T, depth):
    return pl.pallas_call(
        _proj_tc,
        grid=(B, H // RB),
        in_specs=[
            pl.BlockSpec(memory_space=pltpu.SMEM),
            pl.BlockSpec(memory_space=pltpu.SMEM),
            pl.BlockSpec(memory_space=pltpu.SMEM),
            pl.BlockSpec((1, RB, W), lambda b, i: (b, i, 0)),
        ],
        out_specs=pl.BlockSpec((1, RB, W), lambda b, i: (b, i, 0)),
        out_shape=jax.ShapeDtypeStruct((B, H, W), jnp.int32),
    )(A, K, T, depth)


def _sc_body(t_hbm, img_hbm, out_hbm, nmax_v, tbuf_v,
             idx0_v, idx1_v, idx2_v, val0_v, val1_v, val2_v,
             orow0_v, orow1_v, orow2_v, key_v, row_v, gsem):
    idx_v = (idx0_v, idx1_v, idx2_v)
    val_v = (val0_v, val1_v, val2_v)
    orow_v = (orow0_v, orow1_v, orow2_v)
    cid = lax.axis_index("c")
    sid = lax.axis_index("s")
    wid = sid * 2 + cid
    b = wid // NQ
    q = wid % NQ
    lo = q * QCELLS
    lanes = lax.iota(jnp.int32, 16)
    nxt_idx = jnp.minimum(lanes + 1, 15)
    pair_idx = lanes - (lanes & 1)

    def init_body(i, _):
        nmax_v[pl.ds(i * 16, 16)] = jnp.full((16,), -1, jnp.int32)
        return _

    lax.fori_loop(0, QCELLS // 16, init_body, None, unroll=4)

    def chunk_body(ch, _):
        pltpu.sync_copy(t_hbm.at[pl.ds(b * HW + ch * TCH, TCH)], tbuf_v)

        def vec_body(v, __):
            tv = tbuf_v[pl.ds(v * 16, 16)]
            m = (tv >= lo) & (tv < lo + QCELLS)
            plsc.store_scatter(nmax_v, [tv - lo], ch * TCH + v * 16 + lanes, mask=m)
            return __

        lax.fori_loop(0, TCH // 16, vec_body, None)
        return _

    lax.fori_loop(0, HW // TCH, chunk_body, None)

    boff = b * (C * HW)
    row0 = q * QROWS

    def pair_body(p, _):
        r = row0 + 2 * p

        def bld(v, __):
            nm = nmax_v[pl.ds(p * 1024 + v * 16, 16)]
            gcell = lo + p * 1024 + v * 16 + lanes
            safe = jnp.where(nm >= 0, nm, gcell)
            for c in range(C):
                idx_v[c][pl.ds(v * 16, 16)] = safe + (boff + c * HW)
            return __

        lax.fori_loop(0, 64, bld, None)
        cps = [
            pltpu.async_copy(img_hbm.at[idx_v[c]], val_v[c], gsem)
            for c in range(C)
        ]
        for cp in cps:
            cp.wait()

        for c in range(C):
            def vmax_body(j, __, c=c):
                nm_t = nmax_v[pl.ds(p * 1024 + j * 16, 16)]
                nm_b = nmax_v[pl.ds(p * 1024 + 512 + j * 16, 16)]
                top = jnp.where(nm_t >= 0, val_v[c][pl.ds(j * 16, 16)], 0.0)
                bot = jnp.where(nm_b >= 0, val_v[c][pl.ds(512 + j * 16, 16)], 0.0)
                row_v[pl.ds(j * 16, 16)] = jnp.maximum(top, bot)
                return __

            lax.fori_loop(0, W // 16, vmax_body, None)

            def hmax_body(j, __, c=c):
                base = j * 16
                a = plsc.load_gather(row_v, [base + pair_idx])
                bb = plsc.load_gather(row_v, [base + pair_idx + 1])
                orow_v[c][pl.ds(base, 16)] = jnp.maximum(a, bb)
                return __

            lax.fori_loop(0, W // 16, hmax_body, None)

        for c in range(C):
            o = boff + c * HW + r * W
            pltpu.sync_copy(orow_v[c], out_hbm.at[pl.ds(o, W)])
            pltpu.sync_copy(orow_v[c], out_hbm.at[pl.ds(o + W, W)])
        return _

    lax.fori_loop(0, QROWS // 2, pair_body, None)


@functools.cache
def _make_sc_call():
    return pl.kernel(
        _sc_body,
        out_type=jax.ShapeDtypeStruct((B * C * HW,), jnp.float32),
        mesh=plsc.VectorSubcoreMesh(core_axis_name="c", subcore_axis_name="s"),
        compiler_params=pltpu.CompilerParams(needs_layout_passes=False),
        scratch_types=[
        pltpu.VMEM((QCELLS,), jnp.int32),
        pltpu.VMEM((TCH,), jnp.int32),
        pltpu.VMEM((1024,), jnp.int32),
        pltpu.VMEM((1024,), jnp.int32),
        pltpu.VMEM((1024,), jnp.int32),
        pltpu.VMEM((1024,), jnp.float32),
        pltpu.VMEM((1024,), jnp.float32),
        pltpu.VMEM((1024,), jnp.float32),
        pltpu.VMEM((W,), jnp.float32),
        pltpu.VMEM((W,), jnp.float32),
        pltpu.VMEM((W,), jnp.float32),
        pltpu.VMEM((16,), jnp.int32),
        pltpu.VMEM((W,), jnp.float32),
        pltpu.SemaphoreType.DMA,
        ],
    )


def kernel(image, depth, T, R, K, Kinv):
    A = jnp.einsum('ij,bjk,kl->bil', K, R, Kinv)
    t = _project(_bfr(A).reshape(B, 9), _bfr(K), T.reshape(B, 3), depth)
    out = _make_sc_call()(t.reshape(B * HW), image.reshape(B * C * HW))
    return out.reshape(B, C, H, W)


# speedup vs baseline: 54.2332x; 1.1577x over previous
import functools

import jax
import jax.numpy as jnp
from jax import lax
from jax.experimental import pallas as pl
from jax.experimental.pallas import tpu as pltpu
from jax.experimental.pallas import tpu_sc as plsc

B, C, H, W = 8, 3, 512, 512
HW = H * W
RB = 16
NQ = 4
QCELLS = HW // NQ
QROWS = H // NQ
TCH = 8192
_HUGE = 0x7FFFFFFF


def _bfr(x):
    u = lax.bitcast_convert_type(x, jnp.uint32)
    r = (u + jnp.uint32(0x7FFF) + ((u >> 16) & jnp.uint32(1))) & jnp.uint32(0xFFFF0000)
    return lax.bitcast_convert_type(r, jnp.float32)


def _proj_tc(a_ref, k_ref, t3_ref, d_ref, o_ref):
    b = pl.program_id(0)
    i = pl.program_id(1)
    gx = _bfr((i * RB + lax.broadcasted_iota(jnp.int32, (1, RB, W), 1)).astype(jnp.float32))
    gy = _bfr(lax.broadcasted_iota(jnp.int32, (1, RB, W), 2).astype(jnp.float32))
    d = d_ref[...]
    td0 = _bfr(t3_ref[b, 0] / d)
    td1 = _bfr(t3_ref[b, 1] / d)
    td2 = _bfr(t3_ref[b, 2] / d)
    kt0 = (k_ref[0, 0] * td0 + k_ref[0, 1] * td1) + k_ref[0, 2] * td2
    kt1 = (k_ref[1, 0] * td0 + k_ref[1, 1] * td1) + k_ref[1, 2] * td2
    kt2 = (k_ref[2, 0] * td0 + k_ref[2, 1] * td1) + k_ref[2, 2] * td2
    n0 = ((a_ref[b, 0] * gx + a_ref[b, 1] * gy) + a_ref[b, 2]) + kt0
    n1 = ((a_ref[b, 3] * gx + a_ref[b, 4] * gy) + a_ref[b, 5]) + kt1
    dn = ((a_ref[b, 6] * gx + a_ref[b, 7] * gy) + a_ref[b, 8]) + kt2
    p0 = jnp.clip(n0 / dn, 0, H - 1).astype(jnp.int32)
    p1 = jnp.clip(n1 / dn, 0, H - 1).astype(jnp.int32)
    o_ref[...] = p0 * W + p1


def _project(A, K, T, depth):
    return pl.pallas_call(
        _proj_tc,
        grid=(B, H // RB),
        in_specs=[
            pl.BlockSpec(memory_space=pltpu.SMEM),
            pl.BlockSpec(memory_space=pltpu.SMEM),
            pl.BlockSpec(memory_space=pltpu.SMEM),
            pl.BlockSpec((1, RB, W), lambda b, i: (b, i, 0)),
        ],
        out_specs=pl.BlockSpec((1, RB, W), lambda b, i: (b, i, 0)),
        out_shape=jax.ShapeDtypeStruct((B, H, W), jnp.int32),
    )(A, K, T, depth)


def _sc_body(t_hbm, img_hbm, out_hbm, nmax_v, tbuf_v,
             ixa0, ixa1, ixa2, ixb0, ixb1, ixb2,
             va0, va1, va2, vb0, vb1, vb2,
             oa0, oa1, oa2, ob0, ob1, ob2,
             row_v, gsem_a, gsem_b, wsem_a, wsem_b):
    idx_v = ((ixa0, ixa1, ixa2), (ixb0, ixb1, ixb2))
    val_v = ((va0, va1, va2), (vb0, vb1, vb2))
    orow_v = ((oa0, oa1, oa2), (ob0, ob1, ob2))
    gsem = (gsem_a, gsem_b)
    wsem = (wsem_a, wsem_b)
    cid = lax.axis_index("c")
    sid = lax.axis_index("s")
    wid = sid * 2 + cid
    b = wid // NQ
    q = wid % NQ
    lo = q * QCELLS
    lanes = lax.iota(jnp.int32, 16)
    nxt_idx = jnp.minimum(lanes + 1, 15)
    pair_idx = lanes - (lanes & 1)

    def init_body(i, _):
        nmax_v[pl.ds(i * 16, 16)] = jnp.full((16,), -1, jnp.int32)
        return _

    lax.fori_loop(0, QCELLS // 16, init_body, None, unroll=4)

    def chunk_body(ch, _):
        pltpu.sync_copy(t_hbm.at[pl.ds(b * HW + ch * TCH, TCH)], tbuf_v)

        def vec_body(v, __):
            tv = tbuf_v[pl.ds(v * 16, 16)]
            m = (tv >= lo) & (tv < lo + QCELLS)
            plsc.store_scatter(nmax_v, [tv - lo], ch * TCH + v * 16 + lanes, mask=m)
            return __

        lax.fori_loop(0, TCH // 16, vec_body, None)
        return _

    lax.fori_loop(0, HW // TCH, chunk_body, None)

    boff = b * (C * HW)
    row0 = q * QROWS
    NP = QROWS // 2

    def bld_fire(p, s):
        def bldb(v, __):
            nm = nmax_v[pl.ds(p * 1024 + v * 16, 16)]
            gcell = lo + p * 1024 + v * 16 + lanes
            safe = jnp.where(nm >= 0, nm, gcell)
            for c in range(C):
                idx_v[s][c][pl.ds(v * 16, 16)] = safe + (boff + c * HW)
            return __

        lax.fori_loop(0, 64, bldb, None)
        for c in range(C):
            pltpu.async_copy(img_hbm.at[idx_v[s][c]], val_v[s][c], gsem[s])

    def wait_gather(s):
        for c in range(C):
            pltpu.make_async_copy(img_hbm.at[idx_v[s][c]], val_v[s][c], gsem[s]).wait()

    def wait_writes(s):
        for c in range(C):
            pltpu.make_async_copy(orow_v[s][c], out_hbm.at[pl.ds(boff, W)], wsem[s]).wait()
            pltpu.make_async_copy(orow_v[s][c], out_hbm.at[pl.ds(boff, W)], wsem[s]).wait()

    def pool_write(p, s):
        r = row0 + 2 * p
        for c in range(C):
            def vmax_body(j, __, c=c):
                nm_t = nmax_v[pl.ds(p * 1024 + j * 16, 16)]
                nm_b = nmax_v[pl.ds(p * 1024 + 512 + j * 16, 16)]
                top = jnp.where(nm_t >= 0, val_v[s][c][pl.ds(j * 16, 16)], 0.0)
                bot = jnp.where(nm_b >= 0, val_v[s][c][pl.ds(512 + j * 16, 16)], 0.0)
                row_v[pl.ds(j * 16, 16)] = jnp.maximum(top, bot)
                return __

            lax.fori_loop(0, W // 16, vmax_body, None)

            def hmax_body(j, __, c=c):
                base = j * 16
                a = plsc.load_gather(row_v, [base + pair_idx])
                bb = plsc.load_gather(row_v, [base + pair_idx + 1])
                orow_v[s][c][pl.ds(base, 16)] = jnp.maximum(a, bb)
                return __

            lax.fori_loop(0, W // 16, hmax_body, None)
        for c in range(C):
            o = boff + c * HW + r * W
            pltpu.async_copy(orow_v[s][c], out_hbm.at[pl.ds(o, W)], wsem[s])
            pltpu.async_copy(orow_v[s][c], out_hbm.at[pl.ds(o + W, W)], wsem[s])

    bld_fire(0, 0)

    def pair2_body(h, _):
        p0 = 2 * h
        bld_fire(p0 + 1, 1)

        @pl.when(h > 0)
        def _d0():
            wait_writes(0)

        wait_gather(0)
        pool_write(p0, 0)

        @pl.when(h + 1 < NP // 2)
        def _f0():
            bld_fire(p0 + 2, 0)

        @pl.when(h > 0)
        def _d1():
            wait_writes(1)

        wait_gather(1)
        pool_write(p0 + 1, 1)
        return _

    lax.fori_loop(0, NP // 2, pair2_body, None)
    wait_writes(0)
    wait_writes(1)


@functools.cache
def _make_sc_call():
    return pl.kernel(
        _sc_body,
        out_type=jax.ShapeDtypeStruct((B * C * HW,), jnp.float32),
        mesh=plsc.VectorSubcoreMesh(core_axis_name="c", subcore_axis_name="s"),
        compiler_params=pltpu.CompilerParams(needs_layout_passes=False),
        scratch_types=[
        pltpu.VMEM((QCELLS,), jnp.int32),
        pltpu.VMEM((TCH,), jnp.int32),
        ] + [pltpu.VMEM((1024,), jnp.int32)] * 6
          + [pltpu.VMEM((1024,), jnp.float32)] * 6
          + [pltpu.VMEM((W,), jnp.float32)] * 6
          + [
        pltpu.VMEM((W,), jnp.float32),
        pltpu.SemaphoreType.DMA,
        pltpu.SemaphoreType.DMA,
        pltpu.SemaphoreType.DMA,
        pltpu.SemaphoreType.DMA,
        ],
    )


def kernel(image, depth, T, R, K, Kinv):
    A = jnp.einsum('ij,bjk,kl->bil', K, R, Kinv)
    t = _project(_bfr(A).reshape(B, 9), _bfr(K), T.reshape(B, 3), depth)
    out = _make_sc_call()(t.reshape(B * HW), image.reshape(B * C * HW))
    return out.reshape(B, C, H, W)


# speedup vs baseline: 54.5769x; 1.0063x over previous
import functools

import jax
import jax.numpy as jnp
from jax import lax
from jax.experimental import pallas as pl
from jax.experimental.pallas import tpu as pltpu
from jax.experimental.pallas import tpu_sc as plsc

B, C, H, W = 8, 3, 512, 512
HW = H * W
RB = 16
NQ = 4
QCELLS = HW // NQ
QROWS = H // NQ
TCH = 8192
_HUGE = 0x7FFFFFFF


def _bfr(x):
    u = lax.bitcast_convert_type(x, jnp.uint32)
    r = (u + jnp.uint32(0x7FFF) + ((u >> 16) & jnp.uint32(1))) & jnp.uint32(0xFFFF0000)
    return lax.bitcast_convert_type(r, jnp.float32)


def _proj_tc(a_ref, k_ref, t3_ref, d_ref, o_ref):
    b = pl.program_id(0)
    i = pl.program_id(1)
    gx = _bfr((i * RB + lax.broadcasted_iota(jnp.int32, (1, RB, W), 1)).astype(jnp.float32))
    gy = _bfr(lax.broadcasted_iota(jnp.int32, (1, RB, W), 2).astype(jnp.float32))
    d = d_ref[...]
    td0 = _bfr(t3_ref[b, 0] / d)
    td1 = _bfr(t3_ref[b, 1] / d)
    td2 = _bfr(t3_ref[b, 2] / d)
    kt0 = (k_ref[0, 0] * td0 + k_ref[0, 1] * td1) + k_ref[0, 2] * td2
    kt1 = (k_ref[1, 0] * td0 + k_ref[1, 1] * td1) + k_ref[1, 2] * td2
    kt2 = (k_ref[2, 0] * td0 + k_ref[2, 1] * td1) + k_ref[2, 2] * td2
    n0 = ((a_ref[b, 0] * gx + a_ref[b, 1] * gy) + a_ref[b, 2]) + kt0
    n1 = ((a_ref[b, 3] * gx + a_ref[b, 4] * gy) + a_ref[b, 5]) + kt1
    dn = ((a_ref[b, 6] * gx + a_ref[b, 7] * gy) + a_ref[b, 8]) + kt2
    p0 = jnp.clip(n0 / dn, 0, H - 1).astype(jnp.int32)
    p1 = jnp.clip(n1 / dn, 0, H - 1).astype(jnp.int32)
    o_ref[...] = p0 * W + p1


def _project(A, K, T, depth):
    return pl.pallas_call(
        _proj_tc,
        grid=(B, H // RB),
        in_specs=[
            pl.BlockSpec(memory_space=pltpu.SMEM),
            pl.BlockSpec(memory_space=pltpu.SMEM),
            pl.BlockSpec(memory_space=pltpu.SMEM),
            pl.BlockSpec((1, RB, W), lambda b, i: (b, i, 0)),
        ],
        out_specs=pl.BlockSpec((1, RB, W), lambda b, i: (b, i, 0)),
        out_shape=jax.ShapeDtypeStruct((B, H, W), jnp.int32),
    )(A, K, T, depth)


def _sc_body(t_hbm, img_hbm, out_hbm, nmax_v, tbuf_v,
             ixa0, ixa1, ixa2, ixb0, ixb1, ixb2,
             va0, va1, va2, vb0, vb1, vb2,
             oa0, oa1, oa2, ob0, ob1, ob2,
             row_v, gsem_a, gsem_b, wsem_a, wsem_b):
    idx_v = ((ixa0, ixa1, ixa2), (ixb0, ixb1, ixb2))
    val_v = ((va0, va1, va2), (vb0, vb1, vb2))
    orow_v = ((oa0, oa1, oa2), (ob0, ob1, ob2))
    gsem = (gsem_a, gsem_b)
    wsem = (wsem_a, wsem_b)
    cid = lax.axis_index("c")
    sid = lax.axis_index("s")
    wid = sid * 2 + cid
    b = wid // NQ
    q = wid % NQ
    lo = q * QCELLS
    lanes = lax.iota(jnp.int32, 16)
    nxt_idx = jnp.minimum(lanes + 1, 15)
    pair_idx = lanes - (lanes & 1)

    def init_body(i, _):
        nmax_v[pl.ds(i * 16, 16)] = jnp.full((16,), -1, jnp.int32)
        return _

    lax.fori_loop(0, QCELLS // 16, init_body, None, unroll=4)

    def chunk_body(ch, _):
        pltpu.sync_copy(t_hbm.at[pl.ds(b * HW + ch * TCH, TCH)], tbuf_v)

        def vec_body(v, __):
            tv = tbuf_v[pl.ds(v * 16, 16)]
            m = (tv >= lo) & (tv < lo + QCELLS)
            plsc.store_scatter(nmax_v, [tv - lo], ch * TCH + v * 16 + lanes, mask=m)
            return __

        lax.fori_loop(0, TCH // 16, vec_body, None, unroll=8)
        return _

    lax.fori_loop(0, HW // TCH, chunk_body, None)

    boff = b * (C * HW)
    row0 = q * QROWS
    NP = QROWS // 2

    def bld_fire(p, s):
        def bldb(v, __):
            nm = nmax_v[pl.ds(p * 1024 + v * 16, 16)]
            gcell = lo + p * 1024 + v * 16 + lanes
            safe = jnp.where(nm >= 0, nm, gcell)
            for c in range(C):
                idx_v[s][c][pl.ds(v * 16, 16)] = safe + (boff + c * HW)
            return __

        lax.fori_loop(0, 64, bldb, None, unroll=4)
        for c in range(C):
            pltpu.async_copy(img_hbm.at[idx_v[s][c]], val_v[s][c], gsem[s])

    def wait_gather(s):
        for c in range(C):
            pltpu.make_async_copy(img_hbm.at[idx_v[s][c]], val_v[s][c], gsem[s]).wait()

    def wait_writes(s):
        for c in range(C):
            pltpu.make_async_copy(orow_v[s][c], out_hbm.at[pl.ds(boff, W)], wsem[s]).wait()
            pltpu.make_async_copy(orow_v[s][c], out_hbm.at[pl.ds(boff, W)], wsem[s]).wait()

    def pool_write(p, s):
        r = row0 + 2 * p
        for c in range(C):
            def vmax_body(j, __, c=c):
                nm_t = nmax_v[pl.ds(p * 1024 + j * 16, 16)]
                nm_b = nmax_v[pl.ds(p * 1024 + 512 + j * 16, 16)]
                top = jnp.where(nm_t >= 0, val_v[s][c][pl.ds(j * 16, 16)], 0.0)
                bot = jnp.where(nm_b >= 0, val_v[s][c][pl.ds(512 + j * 16, 16)], 0.0)
                row_v[pl.ds(j * 16, 16)] = jnp.maximum(top, bot)
                return __

            lax.fori_loop(0, W // 16, vmax_body, None, unroll=4)

            def hmax_body(j, __, c=c):
                base = j * 16
                a = plsc.load_gather(row_v, [base + pair_idx])
                bb = plsc.load_gather(row_v, [base + pair_idx + 1])
                orow_v[s][c][pl.ds(base, 16)] = jnp.maximum(a, bb)
                return __

            lax.fori_loop(0, W // 16, hmax_body, None, unroll=4)
        for c in range(C):
            o = boff + c * HW + r * W
            pltpu.async_copy(orow_v[s][c], out_hbm.at[pl.ds(o, W)], wsem[s])
            pltpu.async_copy(orow_v[s][c], out_hbm.at[pl.ds(o + W, W)], wsem[s])

    bld_fire(0, 0)

    def pair2_body(h, _):
        p0 = 2 * h
        bld_fire(p0 + 1, 1)

        @pl.when(h > 0)
        def _d0():
            wait_writes(0)

        wait_gather(0)
        pool_write(p0, 0)

        @pl.when(h + 1 < NP // 2)
        def _f0():
            bld_fire(p0 + 2, 0)

        @pl.when(h > 0)
        def _d1():
            wait_writes(1)

        wait_gather(1)
        pool_write(p0 + 1, 1)
        return _

    lax.fori_loop(0, NP // 2, pair2_body, None)
    wait_writes(0)
    wait_writes(1)


@functools.cache
def _make_sc_call():
    return pl.kernel(
        _sc_body,
        out_type=jax.ShapeDtypeStruct((B * C * HW,), jnp.float32),
        mesh=plsc.VectorSubcoreMesh(core_axis_name="c", subcore_axis_name="s"),
        compiler_params=pltpu.CompilerParams(needs_layout_passes=False),
        scratch_types=[
        pltpu.VMEM((QCELLS,), jnp.int32),
        pltpu.VMEM((TCH,), jnp.int32),
        ] + [pltpu.VMEM((1024,), jnp.int32)] * 6
          + [pltpu.VMEM((1024,), jnp.float32)] * 6
          + [pltpu.VMEM((W,), jnp.float32)] * 6
          + [
        pltpu.VMEM((W,), jnp.float32),
        pltpu.SemaphoreType.DMA,
        pltpu.SemaphoreType.DMA,
        pltpu.SemaphoreType.DMA,
        pltpu.SemaphoreType.DMA,
        ],
    )


def kernel(image, depth, T, R, K, Kinv):
    A = jnp.einsum('ij,bjk,kl->bil', K, R, Kinv)
    t = _project(_bfr(A).reshape(B, 9), _bfr(K), T.reshape(B, 3), depth)
    out = _make_sc_call()(t.reshape(B * HW), image.reshape(B * C * HW))
    return out.reshape(B, C, H, W)
